# Initial kernel scaffold; baseline (speedup 1.0000x reference)
import jax
import jax.numpy as jnp
from jax.experimental import pallas as pl


def kernel(mllm_mask, byt5_mask):
    raise NotImplementedError("write your pallas kernel here")


# speedup vs baseline: 1.1464x; 1.1464x over previous
import functools

import jax
import jax.numpy as jnp
from jax import lax
from jax.experimental import pallas as pl
from jax.experimental.pallas import tpu as pltpu
from jax.experimental.pallas import tpu_sc as plsc

N = 1985
PAD = 2048
NCH = PAD // 16


def _partition_body(comb_hbm, idx_hbm, zmask_hbm, comb_v, idx_v, zmask_v):
    @pl.when((lax.axis_index("c") == 0) & (lax.axis_index("s") == 0))
    def _():
        pltpu.sync_copy(comb_hbm, comb_v)

        def count_body(j, acc):
            v = comb_v[pl.ds(j * 16, 16)]
            return acc + jnp.sum(v, axis=0)

        nv_f = lax.fori_loop(0, NCH, count_body, jnp.float32(0.0))
        nv = nv_f.astype(jnp.int32)

        def part_body(j, carry):
            c1, c0 = carry
            v = comb_v[pl.ds(j * 16, 16)]
            pos = lax.iota(jnp.int32, 16) + j * 16
            m1 = v == 1.0
            m0 = v == 0.0
            plsc.store_compressed(idx_v.at[pl.ds(c1, 16)], pos, mask=m1)
            plsc.store_compressed(idx_v.at[pl.ds(nv + c0, 16)], pos, mask=m0)
            zmask_v[pl.ds(j * 16, 16)] = jnp.where(
                pos < nv, jnp.float32(1.0), jnp.float32(0.0)
            )
            c1 = c1 + jnp.sum(m1.astype(jnp.int32), axis=0)
            c0 = c0 + jnp.sum(m0.astype(jnp.int32), axis=0)
            return (c1, c0)

        lax.fori_loop(0, NCH, part_body, (jnp.int32(0), jnp.int32(0)))

        pltpu.sync_copy(idx_v.at[pl.ds(0, PAD)], idx_hbm)
        pltpu.sync_copy(zmask_v, zmask_hbm)


_partition = pl.kernel(
    _partition_body,
    out_type=(
        jax.ShapeDtypeStruct((PAD,), jnp.int32),
        jax.ShapeDtypeStruct((PAD,), jnp.float32),
    ),
    mesh=plsc.VectorSubcoreMesh(core_axis_name="c", subcore_axis_name="s"),
    compiler_params=pltpu.CompilerParams(needs_layout_passes=False),
    scratch_types=[
        pltpu.VMEM((PAD,), jnp.float32),
        pltpu.VMEM((PAD + 16,), jnp.int32),
        pltpu.VMEM((PAD,), jnp.float32),
    ],
)


@jax.jit
def kernel(mllm_mask, byt5_mask):
    combined = jnp.concatenate(
        [
            mllm_mask.astype(jnp.float32),
            byt5_mask.astype(jnp.float32),
            jnp.zeros(PAD - 1256, dtype=jnp.float32),
        ]
    )
    idx, zmask = _partition(combined)
    return idx[:N], zmask[:N]
